# Initial kernel scaffold; baseline (speedup 1.0000x reference)
import jax
import jax.numpy as jnp
from jax.experimental import pallas as pl


def kernel(uid, did, hour, weekday, fresh, position, history, uemb_table, demb_table, hour_table, weekday_table, fresh_hour_table, fresh_day_table, position_table, W1, b1, W2, b2, W3, b3, Wd, bd):
    raise NotImplementedError("write your pallas kernel here")


# speedup vs baseline: 5.7871x; 5.7871x over previous
import functools

import jax
import jax.numpy as jnp
from jax import lax
from jax.experimental import pallas as pl
from jax.experimental.pallas import tpu as pltpu
from jax.experimental.pallas import tpu_sc as plsc

B = 16384
D = 16
L = 50
NC = 2
NS = 16
NW = NC * NS
BPW = B // NW
C = 64
NCHUNK = BPW // C
PIECE = 100
NPIECE = C * L // PIECE


def _sc_features(uid, did, hour, weekday, fresh, position, hist3,
                 uemb_table, demb_table, hour_table, weekday_table,
                 fresh_hour_table, fresh_day_table, position_table):
  mesh = plsc.VectorSubcoreMesh(core_axis_name="c", subcore_axis_name="s")

  @functools.partial(
      pl.kernel,
      out_type=jax.ShapeDtypeStruct((B, 8 * D), jnp.float32),
      mesh=mesh,
      compiler_params=pltpu.CompilerParams(
          needs_layout_passes=False, use_tc_tiling_on_sc=False),
      scratch_types=[
          pltpu.VMEM((8, C), jnp.int32),
          pltpu.VMEM((NPIECE, PIECE), jnp.int32),
          pltpu.VMEM((7, C, D), jnp.float32),
          pltpu.VMEM((NPIECE, PIECE, D), jnp.float32),
          pltpu.VMEM((C, 8 * D), jnp.float32),
          pltpu.VMEM((1, D), jnp.float32),
          pltpu.VMEM((C,), jnp.float32),
          pltpu.VMEM((C,), jnp.float32),
          pltpu.SemaphoreType.DMA,
      ],
  )
  def k(uid_h, did_h, hour_h, wd_h, fresh_h, pos_h, hist3_h,
        uembt_h, dembt_h, hourt_h, wdt_h, fht_h, fdt_h, post_h,
        x_h,
        idx_v, histidx_v, feat_v, histrows_v, xtile_v, row0_v, cnt_v,
        hlinv_v, sem):
    wid = lax.axis_index("s") * NC + lax.axis_index("c")
    pltpu.sync_copy(dembt_h.at[pl.ds(0, 1), :], row0_v)
    row0 = row0_v[0]

    def chunk_body(cidx, carry):
      base = pl.multiple_of(wid * BPW + cidx * C, C)
      hrow = pl.multiple_of(base * L // PIECE, NPIECE)

      pltpu.sync_copy(uid_h.at[pl.ds(base, C)], idx_v.at[0])
      pltpu.sync_copy(did_h.at[pl.ds(base, C)], idx_v.at[1])
      pltpu.sync_copy(hour_h.at[pl.ds(base, C)], idx_v.at[2])
      pltpu.sync_copy(wd_h.at[pl.ds(base, C)], idx_v.at[3])
      pltpu.sync_copy(fresh_h.at[pl.ds(base, C)], idx_v.at[4])
      pltpu.sync_copy(pos_h.at[pl.ds(base, C)], idx_v.at[5])
      pltpu.sync_copy(hist3_h.at[pl.ds(hrow, NPIECE), :], histidx_v)

      for kk in range(C // 16):
        v = idx_v[4, pl.ds(kk * 16, 16)]
        idx_v[6, pl.ds(kk * 16, 16)] = jnp.minimum(lax.div(v, 3600), 299)
        idx_v[7, pl.ds(kk * 16, 16)] = jnp.minimum(lax.div(v, 3600 * 12), 49)

      descs = []
      for p in range(NPIECE):
        descs.append(pltpu.async_copy(
            dembt_h.at[histidx_v.at[p]], histrows_v.at[p], sem))
      for f, (tab, irow) in enumerate([
          (uembt_h, 0), (dembt_h, 1), (hourt_h, 2), (wdt_h, 3),
          (fdt_h, 7), (fht_h, 6), (post_h, 5)]):
        descs.append(pltpu.async_copy(tab.at[idx_v.at[irow]], feat_v.at[f], sem))

      lane = lax.iota(jnp.int32, 16)
      for g in range(C // 16):
        svec = g * 16 + lane
        rvec = lax.div(svec, 2)
        c0 = (svec - rvec * 2) * L
        cnt = jnp.zeros((16,), jnp.int32)
        for j in range(L):
          hv = plsc.load_gather(histidx_v, [rvec, c0 + j])
          cnt = cnt + jnp.where(hv == 0, 1, 0)
        cntf = cnt.astype(jnp.float32)
        cnt_v[pl.ds(g * 16, 16)] = cntf
        hlinv_v[pl.ds(g * 16, 16)] = 1.0 / jnp.maximum(
            jnp.float32(L) - cntf, 1.0)

      for dsc in descs:
        dsc.wait()

      def sample_body(s, carry2):
        srow = lax.div(s, 2)
        soff = (s - srow * 2) * L
        svec = jnp.broadcast_to(s, (16,))
        n0 = plsc.load_gather(cnt_v, [svec])
        hlinv = plsc.load_gather(hlinv_v, [svec])

        accs = [row0 * (-n0), jnp.zeros((16,), jnp.float32),
                jnp.zeros((16,), jnp.float32), jnp.zeros((16,), jnp.float32)]
        for j in range(L):
          accs[j % 4] = accs[j % 4] + histrows_v[srow, soff + j]
        his = (accs[0] + accs[1] + accs[2] + accs[3]) * hlinv

        for f in range(7):
          xtile_v[s, pl.ds(f * D, D)] = feat_v[f, s]
        xtile_v[s, pl.ds(7 * D, D)] = his
        return carry2

      lax.fori_loop(0, C, sample_body, 0)
      pltpu.sync_copy(xtile_v, x_h.at[pl.ds(base, C), :])
      return carry

    lax.fori_loop(0, NCHUNK, chunk_body, 0)

  return k(uid, did, hour, weekday, fresh, position, hist3,
           uemb_table, demb_table, hour_table, weekday_table,
           fresh_hour_table, fresh_day_table, position_table)


def _mlp(x, W1, b1, W2, b2, W3, b3, Wd, bd):
  BLK = 2048

  def body(x_ref, w1_ref, b1_ref, w2_ref, b2_ref, w3_ref, b3_ref,
           wd_ref, bd_ref, out_ref):
    xb = x_ref[...]
    h = jnp.maximum(
        jnp.dot(xb, w1_ref[...], preferred_element_type=jnp.float32)
        + b1_ref[...], 0.0)
    h = jnp.maximum(
        jnp.dot(h, w2_ref[...], preferred_element_type=jnp.float32)
        + b2_ref[...], 0.0)
    h = jnp.maximum(
        jnp.dot(h, w3_ref[...], preferred_element_type=jnp.float32)
        + b3_ref[...], 0.0)
    out_ref[...] = (
        jnp.dot(h, wd_ref[...], preferred_element_type=jnp.float32)
        + bd_ref[...])

  zero = lambda i: (0, 0)
  return pl.pallas_call(
      body,
      grid=(B // BLK,),
      in_specs=[
          pl.BlockSpec((BLK, 128), lambda i: (i, 0)),
          pl.BlockSpec((128, 512), zero),
          pl.BlockSpec((1, 512), zero),
          pl.BlockSpec((512, 256), zero),
          pl.BlockSpec((1, 256), zero),
          pl.BlockSpec((256, 64), zero),
          pl.BlockSpec((1, 64), zero),
          pl.BlockSpec((64, 1), zero),
          pl.BlockSpec((1, 1), zero),
      ],
      out_specs=pl.BlockSpec((BLK, 1), lambda i: (i, 0)),
      out_shape=jax.ShapeDtypeStruct((B, 1), jnp.float32),
  )(x, W1, b1.reshape(1, 512), W2, b2.reshape(1, 256),
    W3, b3.reshape(1, 64), Wd, bd.reshape(1, 1))


def kernel(uid, did, hour, weekday, fresh, position, history,
           uemb_table, demb_table, hour_table, weekday_table,
           fresh_hour_table, fresh_day_table, position_table,
           W1, b1, W2, b2, W3, b3, Wd, bd):
  hist3 = history.reshape(B * L // PIECE, PIECE)
  x = _sc_features(uid, did, hour, weekday, fresh, position, hist3,
                   uemb_table, demb_table, hour_table, weekday_table,
                   fresh_hour_table, fresh_day_table, position_table)
  return _mlp(x, W1, b1, W2, b2, W3, b3, Wd, bd)
